# baseline (speedup 1.0000x reference)
import jax
import jax.numpy as jnp
from jax import lax
from jax.experimental import pallas as pl
from jax.experimental.pallas import tpu as pltpu
from jax.experimental.pallas import tpu_sc as plsc

VOCAB = 1000000
D = 64
DP = 128
B = 4096
S = 200

NC, NS = 2, 16
NW = NC * NS
ROWS_W = B // NW
NBUF = 4
NROUND = ROWS_W // NBUF


def _gather_body(x_hbm, w_hbm, out_hbm, idx_v, rows_v, gsems, osems):
    wid = lax.axis_index("s") * NC + lax.axis_index("c")
    base = wid * ROWS_W
    pltpu.sync_copy(x_hbm.at[pl.ds(base, ROWS_W)], idx_v)

    def start_gather(r, b):
        pltpu.async_copy(w_hbm.at[idx_v.at[r]], rows_v.at[b], gsems.at[b])

    def wait_gather(r, b):
        pltpu.make_async_copy(w_hbm.at[idx_v.at[r]], rows_v.at[b],
                              gsems.at[b]).wait()

    def start_out(r, b):
        pltpu.async_copy(rows_v.at[b, :, pl.ds(0, D)],
                         out_hbm.at[base + r, :, pl.ds(0, D)], osems.at[b])

    def wait_out(r, b):
        pltpu.make_async_copy(rows_v.at[b, :, pl.ds(0, D)],
                              out_hbm.at[base + r, :, pl.ds(0, D)],
                              osems.at[b]).wait()

    for b in range(NBUF):
        start_gather(b, b)

    @pl.loop(0, NROUND - 1)
    def _(t):
        for b in range(NBUF):
            r = t * NBUF + b
            wait_gather(r, b)
            start_out(r, b)
            wait_out(r, b)
            start_gather(r + NBUF, b)

    for b in range(NBUF):
        r = (NROUND - 1) * NBUF + b
        wait_gather(r, b)
        start_out(r, b)
    for b in range(NBUF):
        r = (NROUND - 1) * NBUF + b
        wait_out(r, b)


@jax.jit
def kernel(x, W):
    Wp = jnp.zeros((VOCAB, DP), jnp.float32).at[:, :D].set(W)
    mesh = plsc.VectorSubcoreMesh(
        core_axis_name="c", subcore_axis_name="s",
        num_cores=NC, num_subcores=NS)
    out = pl.kernel(
        _gather_body,
        out_type=jax.ShapeDtypeStruct((B, S, DP), jnp.float32),
        mesh=mesh,
        compiler_params=pltpu.CompilerParams(use_tc_tiling_on_sc=False),
        scratch_types=[
            pltpu.VMEM((ROWS_W, S), jnp.int32),
            pltpu.VMEM((NBUF, S, DP), jnp.float32),
            pltpu.SemaphoreType.DMA((NBUF,)),
            pltpu.SemaphoreType.DMA((NBUF,)),
        ],
    )(x, Wp)
    return out[:, :, :D]


# speedup vs baseline: 1.3387x; 1.3387x over previous
import jax
import jax.numpy as jnp
from jax import lax
from jax.experimental import pallas as pl
from jax.experimental.pallas import tpu as pltpu
from jax.experimental.pallas import tpu_sc as plsc

VOCAB = 1000000
D = 64
DP = 128
B = 4096
S = 200

NC, NS = 2, 16
NW = NC * NS
ROWS_W = B // NW
NBUF = 4
NROUND = ROWS_W // NBUF


def _gather_body(x_hbm, w_hbm, out_hbm, idx_v, rows_v, gsems, osems):
    wid = lax.axis_index("s") * NC + lax.axis_index("c")
    base = wid * ROWS_W
    pltpu.sync_copy(x_hbm.at[pl.ds(base, ROWS_W)], idx_v)

    def start_gather(r, b):
        pltpu.async_copy(w_hbm.at[idx_v.at[r]], rows_v.at[b], gsems.at[b])

    def wait_gather(r, b):
        pltpu.make_async_copy(w_hbm.at[idx_v.at[r]], rows_v.at[b],
                              gsems.at[b]).wait()

    def start_out(r, b):
        pltpu.async_copy(rows_v.at[b, :, pl.ds(0, D)],
                         out_hbm.at[base + r, :, pl.ds(0, D)], osems.at[b])

    def wait_out(r, b):
        pltpu.make_async_copy(rows_v.at[b, :, pl.ds(0, D)],
                              out_hbm.at[base + r, :, pl.ds(0, D)],
                              osems.at[b]).wait()

    for b in range(NBUF):
        start_gather(b, b)

    @pl.loop(0, NROUND - 1)
    def _(t):
        for b in range(NBUF):
            r = t * NBUF + b
            wait_gather(r, b)
            start_out(r, b)
            wait_out(r, b)
            start_gather(r + NBUF, b)

    for b in range(NBUF):
        r = (NROUND - 1) * NBUF + b
        wait_gather(r, b)
        start_out(r, b)
    for b in range(NBUF):
        r = (NROUND - 1) * NBUF + b
        wait_out(r, b)


@jax.jit
def kernel(x, W):
    Wp = jnp.pad(W, ((0, 0), (0, DP - D)))
    mesh = plsc.VectorSubcoreMesh(
        core_axis_name="c", subcore_axis_name="s",
        num_cores=NC, num_subcores=NS)
    out = pl.kernel(
        _gather_body,
        out_type=jax.ShapeDtypeStruct((B, S, DP), jnp.float32),
        mesh=mesh,
        compiler_params=pltpu.CompilerParams(use_tc_tiling_on_sc=False),
        scratch_types=[
            pltpu.VMEM((ROWS_W, S), jnp.int32),
            pltpu.VMEM((NBUF, S, DP), jnp.float32),
            pltpu.SemaphoreType.DMA((NBUF,)),
            pltpu.SemaphoreType.DMA((NBUF,)),
        ],
    )(x, Wp)
    return out[:, :, :D]


# speedup vs baseline: 1.3432x; 1.0033x over previous
import jax
import jax.numpy as jnp
from jax import lax
from jax.experimental import pallas as pl
from jax.experimental.pallas import tpu as pltpu
from jax.experimental.pallas import tpu_sc as plsc

VOCAB = 1000000
D = 64
DP = 128
B = 4096
S = 200

NC, NS = 2, 16
NW = NC * NS
ROWS_W = B // NW
NBUF = 8
HALF = 5
CH = S // HALF
NCHUNK = ROWS_W * HALF
NROUND = NCHUNK // NBUF


def _gather_body(x_hbm, w_hbm, out_hbm, idx_v, rows_v, gsems, osems):
    wid = lax.axis_index("s") * NC + lax.axis_index("c")
    base = wid * ROWS_W
    pltpu.sync_copy(x_hbm.at[pl.ds(base, ROWS_W)], idx_v)

    def start_gather(c, b):
        r, h = c // HALF, c % HALF
        pltpu.async_copy(w_hbm.at[idx_v.at[r, pl.ds(h * CH, CH)]],
                         rows_v.at[b], gsems.at[b])

    def wait_gather(c, b):
        r, h = c // HALF, c % HALF
        pltpu.make_async_copy(w_hbm.at[idx_v.at[r, pl.ds(h * CH, CH)]],
                              rows_v.at[b], gsems.at[b]).wait()

    def start_out(c, b):
        r, h = c // HALF, c % HALF
        pltpu.async_copy(
            rows_v.at[b, :, pl.ds(0, D)],
            out_hbm.at[base + r, pl.ds(h * CH, CH), pl.ds(0, D)], osems.at[b])

    def wait_out(c, b):
        r, h = c // HALF, c % HALF
        pltpu.make_async_copy(
            rows_v.at[b, :, pl.ds(0, D)],
            out_hbm.at[base + r, pl.ds(h * CH, CH), pl.ds(0, D)],
            osems.at[b]).wait()

    for b in range(NBUF):
        start_gather(b, b)

    @pl.loop(0, NROUND - 1)
    def _(t):
        for b in range(NBUF):
            r = t * NBUF + b
            wait_gather(r, b)
            start_out(r, b)
            wait_out(r, b)
            start_gather(r + NBUF, b)

    for b in range(NBUF):
        r = (NROUND - 1) * NBUF + b
        wait_gather(r, b)
        start_out(r, b)
    for b in range(NBUF):
        r = (NROUND - 1) * NBUF + b
        wait_out(r, b)


@jax.jit
def kernel(x, W):
    Wp = jnp.pad(W, ((0, 0), (0, DP - D)))
    mesh = plsc.VectorSubcoreMesh(
        core_axis_name="c", subcore_axis_name="s",
        num_cores=NC, num_subcores=NS)
    out = pl.kernel(
        _gather_body,
        out_type=jax.ShapeDtypeStruct((B, S, DP), jnp.float32),
        mesh=mesh,
        compiler_params=pltpu.CompilerParams(use_tc_tiling_on_sc=False),
        scratch_types=[
            pltpu.VMEM((ROWS_W, S), jnp.int32),
            pltpu.VMEM((NBUF, CH, DP), jnp.float32),
            pltpu.SemaphoreType.DMA((NBUF,)),
            pltpu.SemaphoreType.DMA((NBUF,)),
        ],
    )(x, Wp)
    return out[:, :, :D]
